# baseline (speedup 1.0000x reference)
import functools
import jax
import jax.numpy as jnp
from jax import lax
from jax.experimental import pallas as pl
from jax.experimental.pallas import tpu as pltpu
from jax.experimental.pallas import tpu_sc as plsc

_K = 1024
_D = 256
_N = 8192
_BM = 2048
_NB = _N // _BM
_E_WEIGHT = 0.25
_MANAGE_WEIGHT = 0.1

_info = plsc.get_sparse_core_info()
_NC, _NS = _info.num_cores, _info.num_subcores
_NW = _NC * _NS
_BW = _N // _NW


def _vq_body(x_ref, emb_ref, idx_ref, stats_ref, sump_ref, acc_ref):
    i = pl.program_id(0)

    @pl.when(i == 0)
    def _init():
        sump_ref[...] = jnp.zeros_like(sump_ref)
        acc_ref[0] = 0.0
        acc_ref[1] = 0.0

    x = x_ref[...].reshape(_BM, _D)
    emb = emb_ref[...]
    norm = jnp.sqrt(jnp.sum(x * x, axis=1, keepdims=True))
    xn = x / jnp.maximum(norm, 1e-12)
    s = jnp.sum(xn * xn, axis=1, keepdims=True)
    e2 = jnp.sum(emb * emb, axis=1)
    xm2 = xn * (-2.0)
    dots2 = lax.dot_general(xm2, emb, (((1,), (1,)), ((), ())))
    d = (s + e2[None, :]) + dots2

    ex = jnp.exp(d)
    se = jnp.sum(ex, axis=1, keepdims=True)
    exd = jnp.sum(ex * d, axis=1, keepdims=True)
    r = 1.0 / se
    ent_col = jnp.log(se) - exd * r

    mind = jnp.min(d, axis=1, keepdims=True)
    kiof = lax.broadcasted_iota(jnp.int32, (1, _K), 1).astype(jnp.float32)
    first = jnp.min(jnp.where(d == mind, kiof, float(_K)),
                    axis=1, keepdims=True)
    idx_ref[...] = jnp.broadcast_to(first.astype(jnp.int32), (_BM, 128))

    sump_ref[...] += lax.dot_general(r, ex, (((0,), (0,)), ((), ())))
    acc_ref[0] += jnp.sum(ent_col)
    acc_ref[1] += jnp.sum(mind)

    @pl.when(i == _NB - 1)
    def _fin():
        intra = acc_ref[0] / _N
        mse = acc_ref[1] / (_N * _D)
        avg_p = sump_ref[...] / _N
        inter = jnp.sum(avg_p * jnp.log(avg_p + 1e-8))
        lane = lax.broadcasted_iota(jnp.int32, (1, 128), 1)
        stats_ref[...] = jnp.where(
            lane == 0, intra,
            jnp.where(lane == 1, inter, jnp.where(lane == 2, mse, 0.0)))


def _sc_gather_body(emb_hbm, idx_hbm, out_hbm, idx_v, rows_v, sem):
    wid = lax.axis_index("s") * _NC + lax.axis_index("c")
    base = wid * _BW
    pltpu.sync_copy(idx_hbm.at[pl.ds(base, _BW)], idx_v)
    pltpu.async_copy(emb_hbm.at[idx_v], rows_v, sem).wait()
    pltpu.sync_copy(rows_v, out_hbm.at[pl.ds(base, _BW)])


_sc_gather = functools.partial(
    pl.kernel,
    mesh=plsc.VectorSubcoreMesh(core_axis_name="c", subcore_axis_name="s"),
    out_type=jax.ShapeDtypeStruct((_N, _D), jnp.float32),
    scratch_types=[
        pltpu.VMEM((_BW,), jnp.int32),
        pltpu.VMEM((_BW, _D), jnp.float32),
        pltpu.SemaphoreType.DMA,
    ],
)(_sc_gather_body)


def kernel(inputs, emb_weight):
    xb = jnp.transpose(inputs, (0, 2, 3, 1))
    idxb, stats = pl.pallas_call(
        _vq_body,
        grid=(_NB,),
        in_specs=[
            pl.BlockSpec((_BM // 1024, 32, 32, _D), lambda i: (i, 0, 0, 0)),
            pl.BlockSpec((_K, _D), lambda i: (0, 0)),
        ],
        out_specs=[
            pl.BlockSpec((_BM, 128), lambda i: (i, 0)),
            pl.BlockSpec((1, 128), lambda i: (0, 0)),
        ],
        out_shape=[
            jax.ShapeDtypeStruct((_N, 128), jnp.int32),
            jax.ShapeDtypeStruct((1, 128), jnp.float32),
        ],
        scratch_shapes=[
            pltpu.VMEM((1, _K), jnp.float32),
            pltpu.SMEM((2,), jnp.float32),
        ],
        compiler_params=pltpu.CompilerParams(
            dimension_semantics=("arbitrary",)),
    )(xb, emb_weight)
    idx = idxb[:, 0]
    q = _sc_gather(emb_weight, idx)
    intra = stats[0, 0]
    inter = stats[0, 1]
    mse = stats[0, 2]
    loss = (mse + _E_WEIGHT * mse) + _MANAGE_WEIGHT * (intra + inter)
    out = jnp.transpose(q.reshape(8, 32, 32, _D), (0, 3, 1, 2))
    return (loss, out, mse, mse, intra, inter)


# speedup vs baseline: 1.7559x; 1.7559x over previous
import jax
import jax.numpy as jnp
from jax import lax
from jax.experimental import pallas as pl
from jax.experimental.pallas import tpu as pltpu

_K = 1024
_D = 256
_N = 8192
_BM = 4096
_NB = _N // _BM
_E_WEIGHT = 0.25
_MANAGE_WEIGHT = 0.1


def _vq_body(x_ref, emb_ref, q_ref, stats_ref, sump_ref, acc_ref):
    i = pl.program_id(0)

    @pl.when(i == 0)
    def _init():
        sump_ref[...] = jnp.zeros_like(sump_ref)
        acc_ref[0] = 0.0
        acc_ref[1] = 0.0

    x = x_ref[...].reshape(_BM, _D)
    emb = emb_ref[...]
    norm = jnp.sqrt(jnp.sum(x * x, axis=1, keepdims=True))
    xn = x / jnp.maximum(norm, 1e-12)
    s = jnp.sum(xn * xn, axis=1, keepdims=True)
    e2 = jnp.sum(emb * emb, axis=1)
    xm2 = xn * (-2.0)
    dots2 = lax.dot_general(xm2, emb, (((1,), (1,)), ((), ())))
    d = (s + e2[None, :]) + dots2

    ex = jnp.exp(d)
    se = jnp.sum(ex, axis=1, keepdims=True)
    exd = jnp.sum(ex * d, axis=1, keepdims=True)
    r = 1.0 / se
    ent_col = jnp.log(se) - exd * r

    mind = jnp.min(d, axis=1, keepdims=True)
    kiof = lax.broadcasted_iota(jnp.int32, (1, _K), 1).astype(jnp.float32)
    first = jnp.min(jnp.where(d == mind, kiof, float(_K)),
                    axis=1, keepdims=True)
    oh = (kiof == first).astype(jnp.float32)
    qt = lax.dot_general(oh, emb, (((1,), (0,)), ((), ())))
    q_ref[...] = qt.reshape(_BM // 1024, 32, 32, _D)

    sump_ref[...] += lax.dot_general(r, ex, (((0,), (0,)), ((), ())))
    acc_ref[0] += jnp.sum(ent_col)
    acc_ref[1] += jnp.sum(mind)

    @pl.when(i == _NB - 1)
    def _fin():
        intra = acc_ref[0] / _N
        mse = acc_ref[1] / (_N * _D)
        avg_p = sump_ref[...] / _N
        inter = jnp.sum(avg_p * jnp.log(avg_p + 1e-8))
        lane = lax.broadcasted_iota(jnp.int32, (1, 128), 1)
        stats_ref[...] = jnp.where(
            lane == 0, intra,
            jnp.where(lane == 1, inter, jnp.where(lane == 2, mse, 0.0)))


def kernel(inputs, emb_weight):
    xb = jnp.transpose(inputs, (0, 2, 3, 1))
    q4, stats = pl.pallas_call(
        _vq_body,
        grid=(_NB,),
        in_specs=[
            pl.BlockSpec((_BM // 1024, 32, 32, _D), lambda i: (i, 0, 0, 0)),
            pl.BlockSpec((_K, _D), lambda i: (0, 0)),
        ],
        out_specs=[
            pl.BlockSpec((_BM // 1024, 32, 32, _D), lambda i: (i, 0, 0, 0)),
            pl.BlockSpec((1, 128), lambda i: (0, 0)),
        ],
        out_shape=[
            jax.ShapeDtypeStruct((8, 32, 32, _D), jnp.float32),
            jax.ShapeDtypeStruct((1, 128), jnp.float32),
        ],
        scratch_shapes=[
            pltpu.VMEM((1, _K), jnp.float32),
            pltpu.SMEM((2,), jnp.float32),
        ],
        compiler_params=pltpu.CompilerParams(
            dimension_semantics=("arbitrary",)),
    )(xb, emb_weight)
    intra = stats[0, 0]
    inter = stats[0, 1]
    mse = stats[0, 2]
    loss = (mse + _E_WEIGHT * mse) + _MANAGE_WEIGHT * (intra + inter)
    out = jnp.transpose(q4, (0, 3, 1, 2))
    return (loss, out, mse, mse, intra, inter)


# speedup vs baseline: 1.7688x; 1.0073x over previous
import jax
import jax.numpy as jnp
from jax import lax
from jax.experimental import pallas as pl
from jax.experimental.pallas import tpu as pltpu

_K = 1024
_D = 256
_N = 8192
_BM = 2048
_NB = _N // _BM
_E_WEIGHT = 0.25
_MANAGE_WEIGHT = 0.1


def _vq_body(x_ref, emb_ref, q_ref, stats_ref, sump_ref, acc_ref):
    i = pl.program_id(0)

    @pl.when(i == 0)
    def _init():
        sump_ref[...] = jnp.zeros_like(sump_ref)
        acc_ref[0] = 0.0
        acc_ref[1] = 0.0

    x = x_ref[...].reshape(_BM, _D)
    emb = emb_ref[...]
    norm = jnp.sqrt(jnp.sum(x * x, axis=1, keepdims=True))
    xn = x / jnp.maximum(norm, 1e-12)
    s = jnp.sum(xn * xn, axis=1, keepdims=True)
    e2 = jnp.sum(emb * emb, axis=1)
    xm2 = xn * (-2.0)
    dots2 = lax.dot_general(xm2, emb, (((1,), (1,)), ((), ())))
    d = (s + e2[None, :]) + dots2

    ex = jnp.exp(d)
    se = jnp.sum(ex, axis=1, keepdims=True)
    exd = jnp.sum(ex * d, axis=1, keepdims=True)
    r = 1.0 / se
    ent_col = jnp.log(se) - exd * r

    mind = jnp.min(d, axis=1, keepdims=True)
    kiof = lax.broadcasted_iota(jnp.int32, (1, _K), 1).astype(jnp.float32)
    first = jnp.min(jnp.where(d == mind, kiof, float(_K)),
                    axis=1, keepdims=True)
    oh = (kiof == first).astype(jnp.float32)
    qt = lax.dot_general(oh, emb, (((1,), (0,)), ((), ())))
    q_ref[...] = qt.reshape(_BM // 1024, 32, 32, _D)

    sump_ref[...] += lax.dot_general(r, ex, (((0,), (0,)), ((), ())))
    acc_ref[0] += jnp.sum(ent_col)
    acc_ref[1] += jnp.sum(mind)

    @pl.when(i == _NB - 1)
    def _fin():
        intra = acc_ref[0] / _N
        mse = acc_ref[1] / (_N * _D)
        avg_p = sump_ref[...] / _N
        inter = jnp.sum(avg_p * jnp.log(avg_p + 1e-8))
        lane = lax.broadcasted_iota(jnp.int32, (1, 128), 1)
        stats_ref[...] = jnp.where(
            lane == 0, intra,
            jnp.where(lane == 1, inter, jnp.where(lane == 2, mse, 0.0)))


def kernel(inputs, emb_weight):
    xb = jnp.transpose(inputs, (0, 2, 3, 1))
    q4, stats = pl.pallas_call(
        _vq_body,
        grid=(_NB,),
        in_specs=[
            pl.BlockSpec((_BM // 1024, 32, 32, _D), lambda i: (i, 0, 0, 0)),
            pl.BlockSpec((_K, _D), lambda i: (0, 0)),
        ],
        out_specs=[
            pl.BlockSpec((_BM // 1024, 32, 32, _D), lambda i: (i, 0, 0, 0)),
            pl.BlockSpec((1, 128), lambda i: (0, 0)),
        ],
        out_shape=[
            jax.ShapeDtypeStruct((8, 32, 32, _D), jnp.float32),
            jax.ShapeDtypeStruct((1, 128), jnp.float32),
        ],
        scratch_shapes=[
            pltpu.VMEM((1, _K), jnp.float32),
            pltpu.SMEM((2,), jnp.float32),
        ],
        compiler_params=pltpu.CompilerParams(
            dimension_semantics=("arbitrary",)),
    )(xb, emb_weight)
    intra = stats[0, 0]
    inter = stats[0, 1]
    mse = stats[0, 2]
    loss = (mse + _E_WEIGHT * mse) + _MANAGE_WEIGHT * (intra + inter)
    out = jnp.transpose(q4, (0, 3, 1, 2))
    return (loss, out, mse, mse, intra, inter)


# speedup vs baseline: 1.8041x; 1.0200x over previous
import jax
import jax.numpy as jnp
from jax import lax
from jax.experimental import pallas as pl
from jax.experimental.pallas import tpu as pltpu

_K = 1024
_D = 256
_N = 8192
_BM = 2048
_NB = _N // _BM
_E_WEIGHT = 0.25
_MANAGE_WEIGHT = 0.1


def _vq_body(x_ref, emb_ref, q_ref, stats_ref, sump_ref, acc_ref,
             e2_ref, em2_ref):
    i = pl.program_id(0)

    @pl.when(i == 0)
    def _init():
        sump_ref[...] = jnp.zeros_like(sump_ref)
        acc_ref[0] = 0.0
        acc_ref[1] = 0.0
        emb0 = emb_ref[...]
        e2_ref[...] = jnp.sum(emb0 * emb0, axis=1)[None, :]
        em2_ref[...] = emb0 * (-2.0)

    x = x_ref[...].reshape(_BM, _D)
    emb = emb_ref[...]
    norm = jnp.sqrt(jnp.sum(x * x, axis=1, keepdims=True))
    xn = x / jnp.maximum(norm, 1e-12)
    s = jnp.sum(xn * xn, axis=1, keepdims=True)
    dots2 = lax.dot_general(xn, em2_ref[...], (((1,), (1,)), ((), ())))
    d = (s + e2_ref[...]) + dots2

    ex = jnp.exp(d)
    se = jnp.sum(ex, axis=1, keepdims=True)
    exd = jnp.sum(ex * d, axis=1, keepdims=True)
    r = 1.0 / se
    ent_col = jnp.log(se) - exd * r

    mind = jnp.min(d, axis=1, keepdims=True)
    kiof = lax.broadcasted_iota(jnp.int32, (1, _K), 1).astype(jnp.float32)
    first = jnp.min(jnp.where(d == mind, kiof, float(_K)),
                    axis=1, keepdims=True)
    oh = (kiof == first).astype(jnp.float32)
    qt = lax.dot_general(oh, emb, (((1,), (0,)), ((), ())))
    q_ref[...] = qt.reshape(_BM // 1024, 32, 32, _D)

    sump_ref[...] += lax.dot_general(r, ex, (((0,), (0,)), ((), ())))
    acc_ref[0] += jnp.sum(ent_col)
    acc_ref[1] += jnp.sum(mind)

    @pl.when(i == _NB - 1)
    def _fin():
        intra = acc_ref[0] / _N
        mse = acc_ref[1] / (_N * _D)
        avg_p = sump_ref[...] / _N
        inter = jnp.sum(avg_p * jnp.log(avg_p + 1e-8))
        lane = lax.broadcasted_iota(jnp.int32, (1, 128), 1)
        stats_ref[...] = jnp.where(
            lane == 0, intra,
            jnp.where(lane == 1, inter, jnp.where(lane == 2, mse, 0.0)))


def kernel(inputs, emb_weight):
    xb = jnp.transpose(inputs, (0, 2, 3, 1))
    q4, stats = pl.pallas_call(
        _vq_body,
        grid=(_NB,),
        in_specs=[
            pl.BlockSpec((_BM // 1024, 32, 32, _D), lambda i: (i, 0, 0, 0)),
            pl.BlockSpec((_K, _D), lambda i: (0, 0)),
        ],
        out_specs=[
            pl.BlockSpec((_BM // 1024, 32, 32, _D), lambda i: (i, 0, 0, 0)),
            pl.BlockSpec((1, 128), lambda i: (0, 0)),
        ],
        out_shape=[
            jax.ShapeDtypeStruct((8, 32, 32, _D), jnp.float32),
            jax.ShapeDtypeStruct((1, 128), jnp.float32),
        ],
        scratch_shapes=[
            pltpu.VMEM((1, _K), jnp.float32),
            pltpu.SMEM((2,), jnp.float32),
            pltpu.VMEM((1, _K), jnp.float32),
            pltpu.VMEM((_K, _D), jnp.float32),
        ],
        compiler_params=pltpu.CompilerParams(
            dimension_semantics=("arbitrary",)),
    )(xb, emb_weight)
    intra = stats[0, 0]
    inter = stats[0, 1]
    mse = stats[0, 2]
    loss = (mse + _E_WEIGHT * mse) + _MANAGE_WEIGHT * (intra + inter)
    out = jnp.transpose(q4, (0, 3, 1, 2))
    return (loss, out, mse, mse, intra, inter)
